# baseline (speedup 1.0000x reference)
import jax
import jax.numpy as jnp
from jax.experimental import pallas as pl
from jax.experimental.pallas import tpu as pltpu

NCELL = 4
VOCAB = 16
EMB = 512
HID = 512
BATCH = 64
SEQ = 64
G4 = 4 * HID


def _prep_kernel(emb_ref, wih_ref, bih_ref, bhh_ref, whh_ref,
                 table_ref, wbf_ref):
    vrow = jax.lax.broadcasted_iota(jnp.int32, (VOCAB, 1), 0)
    acc = jnp.zeros((VOCAB, G4), dtype=jnp.float32)
    for j in range(NCELL):
        gj = jax.lax.dot_general(
            emb_ref[...], wih_ref[j],
            (((1,), (1,)), ((), ())),
            preferred_element_type=jnp.float32,
        ) + bih_ref[j][None, :] + bhh_ref[j][None, :]
        mask = (vrow % NCELL == j).astype(jnp.float32)
        acc = acc + mask * gj
    table_ref[...] = acc
    wbf_ref[...] = whh_ref[...].astype(jnp.bfloat16)


def _lstm_kernel(tok_ref, table_ref, wbf_ref, out_ref, h_ref, c_ref):
    t = pl.program_id(0)

    @pl.when(t == 0)
    def _init():
        h_ref[...] = jnp.zeros((BATCH, HID), jnp.float32)
        c_ref[...] = jnp.zeros((BATCH, HID), jnp.float32)

    tok = tok_ref[0]
    vcol = jax.lax.broadcasted_iota(jnp.int32, (BATCH, VOCAB), 1)
    onehot = (tok == vcol).astype(jnp.float32)
    gates = jnp.dot(onehot, table_ref[...], preferred_element_type=jnp.float32)

    assign = tok % NCELL
    h = h_ref[...]
    for j in range(NCELL):
        hj = (h * (assign == j).astype(jnp.float32)).astype(jnp.bfloat16)
        gates = gates + jax.lax.dot_general(
            hj, wbf_ref[j],
            (((1,), (1,)), ((), ())),
            preferred_element_type=jnp.float32,
        )

    i_g = jax.nn.sigmoid(gates[:, 0 * HID:1 * HID])
    f_g = jax.nn.sigmoid(gates[:, 1 * HID:2 * HID])
    g_g = jnp.tanh(gates[:, 2 * HID:3 * HID])
    o_g = jax.nn.sigmoid(gates[:, 3 * HID:4 * HID])
    c_new = f_g * c_ref[...] + i_g * g_g
    h_new = o_g * jnp.tanh(c_new)
    h_ref[...] = h_new
    c_ref[...] = c_new

    @pl.when(t == SEQ - 1)
    def _emit():
        out_ref[:, 0:HID] = h_new
        out_ref[:, HID:2 * HID] = c_new


@jax.jit
def kernel(input, emb, W_ih, W_hh, b_ih, b_hh):
    tokens = jnp.swapaxes(input.astype(jnp.int32), 0, 1).reshape(SEQ, BATCH, 1)

    table, wbf = pl.pallas_call(
        _prep_kernel,
        out_shape=(
            jax.ShapeDtypeStruct((VOCAB, G4), jnp.float32),
            jax.ShapeDtypeStruct((NCELL, G4, HID), jnp.bfloat16),
        ),
    )(emb, W_ih, b_ih, b_hh, W_hh)

    out = pl.pallas_call(
        _lstm_kernel,
        grid=(SEQ,),
        in_specs=[
            pl.BlockSpec((1, BATCH, 1), lambda t: (t, 0, 0)),
            pl.BlockSpec((VOCAB, G4), lambda t: (0, 0)),
            pl.BlockSpec((NCELL, G4, HID), lambda t: (0, 0, 0)),
        ],
        out_specs=pl.BlockSpec((BATCH, 2 * HID), lambda t: (0, 0)),
        out_shape=jax.ShapeDtypeStruct((BATCH, 2 * HID), jnp.float32),
        scratch_shapes=[
            pltpu.VMEM((BATCH, HID), jnp.float32),
            pltpu.VMEM((BATCH, HID), jnp.float32),
        ],
    )(tokens, table, wbf)
    return out


# speedup vs baseline: 1.4254x; 1.4254x over previous
import jax
import jax.numpy as jnp
from jax.experimental import pallas as pl
from jax.experimental.pallas import tpu as pltpu

NCELL = 4
VOCAB = 16
EMB = 512
HID = 512
BATCH = 64
SEQ = 64
G4 = 4 * HID


def _prep_kernel(emb_ref, wih_ref, bih_ref, bhh_ref, whh_ref,
                 table_ref, wbf_ref):
    vrow = jax.lax.broadcasted_iota(jnp.int32, (VOCAB, 1), 0)
    acc = jnp.zeros((VOCAB, G4), dtype=jnp.float32)
    for j in range(NCELL):
        gj = jax.lax.dot_general(
            emb_ref[...], wih_ref[j],
            (((1,), (1,)), ((), ())),
            preferred_element_type=jnp.float32,
        ) + bih_ref[j][None, :] + bhh_ref[j][None, :]
        mask = (vrow % NCELL == j).astype(jnp.float32)
        acc = acc + mask * gj
    table_ref[...] = acc
    for j in range(NCELL):
        wbf_ref[pl.ds(j * HID, HID), :] = jnp.transpose(
            whh_ref[j]).astype(jnp.bfloat16)


def _lstm_kernel(tok_ref, table_ref, wbf_ref, out_ref, h_ref, c_ref):
    t = pl.program_id(0)

    @pl.when(t == 0)
    def _init():
        h_ref[...] = jnp.zeros((BATCH, HID), jnp.float32)
        c_ref[...] = jnp.zeros((BATCH, HID), jnp.float32)

    tok = tok_ref[0]
    vcol = jax.lax.broadcasted_iota(jnp.int32, (BATCH, VOCAB), 1)
    onehot = (tok == vcol).astype(jnp.float32)
    gates = jnp.dot(onehot, table_ref[...], preferred_element_type=jnp.float32)

    assign = tok % NCELL
    h = h_ref[...]
    hbig = jnp.concatenate(
        [h * (assign == j).astype(jnp.float32) for j in range(NCELL)], axis=1
    ).astype(jnp.bfloat16)
    gates = gates + jnp.dot(hbig, wbf_ref[...],
                            preferred_element_type=jnp.float32)

    i_g = jax.nn.sigmoid(gates[:, 0 * HID:1 * HID])
    f_g = jax.nn.sigmoid(gates[:, 1 * HID:2 * HID])
    g_g = jnp.tanh(gates[:, 2 * HID:3 * HID])
    o_g = jax.nn.sigmoid(gates[:, 3 * HID:4 * HID])
    c_new = f_g * c_ref[...] + i_g * g_g
    h_new = o_g * jnp.tanh(c_new)
    h_ref[...] = h_new
    c_ref[...] = c_new

    @pl.when(t == SEQ - 1)
    def _emit():
        out_ref[:, 0:HID] = h_new
        out_ref[:, HID:2 * HID] = c_new


@jax.jit
def kernel(input, emb, W_ih, W_hh, b_ih, b_hh):
    tokens = jnp.swapaxes(input.astype(jnp.int32), 0, 1).reshape(SEQ, BATCH, 1)

    table, wbf = pl.pallas_call(
        _prep_kernel,
        out_shape=(
            jax.ShapeDtypeStruct((VOCAB, G4), jnp.float32),
            jax.ShapeDtypeStruct((NCELL * HID, G4), jnp.bfloat16),
        ),
    )(emb, W_ih, b_ih, b_hh, W_hh)

    out = pl.pallas_call(
        _lstm_kernel,
        grid=(SEQ,),
        in_specs=[
            pl.BlockSpec((1, BATCH, 1), lambda t: (t, 0, 0)),
            pl.BlockSpec((VOCAB, G4), lambda t: (0, 0)),
            pl.BlockSpec((NCELL * HID, G4), lambda t: (0, 0)),
        ],
        out_specs=pl.BlockSpec((BATCH, 2 * HID), lambda t: (0, 0)),
        out_shape=jax.ShapeDtypeStruct((BATCH, 2 * HID), jnp.float32),
        scratch_shapes=[
            pltpu.VMEM((BATCH, HID), jnp.float32),
            pltpu.VMEM((BATCH, HID), jnp.float32),
        ],
    )(tokens, table, wbf)
    return out


# speedup vs baseline: 1.4851x; 1.0419x over previous
import jax
import jax.numpy as jnp
from jax.experimental import pallas as pl
from jax.experimental.pallas import tpu as pltpu

NCELL = 4
VOCAB = 16
EMB = 512
HID = 512
BATCH = 64
SEQ = 64
G4 = 4 * HID


def _prep_kernel(emb_ref, wih_ref, bih_ref, bhh_ref, whh_ref,
                 table_ref, wbf_ref):
    vrow = jax.lax.broadcasted_iota(jnp.int32, (VOCAB, 1), 0)
    acc = jnp.zeros((VOCAB, G4), dtype=jnp.float32)
    for j in range(NCELL):
        gj = jax.lax.dot_general(
            emb_ref[...], wih_ref[j],
            (((1,), (1,)), ((), ())),
            preferred_element_type=jnp.float32,
        ) + bih_ref[j][None, :] + bhh_ref[j][None, :]
        mask = (vrow % NCELL == j).astype(jnp.float32)
        acc = acc + mask * gj
    table_ref[...] = acc
    for j in range(NCELL):
        wbf_ref[pl.ds(j * HID, HID), :] = jnp.transpose(
            whh_ref[j]).astype(jnp.bfloat16)


UNROLL = 4


def _lstm_kernel(tok_ref, table_ref, wbf_ref, out_ref, h_ref, c_ref):
    t = pl.program_id(0)

    @pl.when(t == 0)
    def _init():
        h_ref[...] = jnp.zeros((BATCH, HID), jnp.float32)
        c_ref[...] = jnp.zeros((BATCH, HID), jnp.float32)

    h = h_ref[...]
    c = c_ref[...]
    vcol = jax.lax.broadcasted_iota(jnp.int32, (BATCH, VOCAB), 1)
    for u in range(UNROLL):
        tok = tok_ref[u]
        onehot = (tok == vcol).astype(jnp.float32)
        gates = jnp.dot(onehot, table_ref[...],
                        preferred_element_type=jnp.float32)

        assign = tok % NCELL
        hbig = jnp.concatenate(
            [h * (assign == j).astype(jnp.float32) for j in range(NCELL)],
            axis=1).astype(jnp.bfloat16)
        gates = gates + jnp.dot(hbig, wbf_ref[...],
                                preferred_element_type=jnp.float32)

        i_g = jax.nn.sigmoid(gates[:, 0 * HID:1 * HID])
        f_g = jax.nn.sigmoid(gates[:, 1 * HID:2 * HID])
        g_g = jnp.tanh(gates[:, 2 * HID:3 * HID])
        o_g = jax.nn.sigmoid(gates[:, 3 * HID:4 * HID])
        c = f_g * c + i_g * g_g
        h = o_g * jnp.tanh(c)
    h_ref[...] = h
    c_ref[...] = c

    @pl.when(t == SEQ // UNROLL - 1)
    def _emit():
        out_ref[:, 0:HID] = h
        out_ref[:, HID:2 * HID] = c


@jax.jit
def kernel(input, emb, W_ih, W_hh, b_ih, b_hh):
    tokens = jnp.swapaxes(input.astype(jnp.int32), 0, 1).reshape(SEQ, BATCH, 1)

    table, wbf = pl.pallas_call(
        _prep_kernel,
        out_shape=(
            jax.ShapeDtypeStruct((VOCAB, G4), jnp.float32),
            jax.ShapeDtypeStruct((NCELL * HID, G4), jnp.bfloat16),
        ),
    )(emb, W_ih, b_ih, b_hh, W_hh)

    out = pl.pallas_call(
        _lstm_kernel,
        grid=(SEQ // UNROLL,),
        in_specs=[
            pl.BlockSpec((UNROLL, BATCH, 1), lambda t: (t, 0, 0)),
            pl.BlockSpec((VOCAB, G4), lambda t: (0, 0)),
            pl.BlockSpec((NCELL * HID, G4), lambda t: (0, 0)),
        ],
        out_specs=pl.BlockSpec((BATCH, 2 * HID), lambda t: (0, 0)),
        out_shape=jax.ShapeDtypeStruct((BATCH, 2 * HID), jnp.float32),
        scratch_shapes=[
            pltpu.VMEM((BATCH, HID), jnp.float32),
            pltpu.VMEM((BATCH, HID), jnp.float32),
        ],
    )(tokens, table, wbf)
    return out


# speedup vs baseline: 1.5015x; 1.0110x over previous
import jax
import jax.numpy as jnp
from jax.experimental import pallas as pl
from jax.experimental.pallas import tpu as pltpu

NCELL = 4
VOCAB = 16
EMB = 512
HID = 512
BATCH = 64
SEQ = 64
G4 = 4 * HID


def _prep_kernel(emb_ref, wih_ref, bih_ref, bhh_ref, whh_ref,
                 table_ref, wbf_ref):
    vrow = jax.lax.broadcasted_iota(jnp.int32, (VOCAB, 1), 0)
    acc = jnp.zeros((VOCAB, G4), dtype=jnp.float32)
    for j in range(NCELL):
        gj = jax.lax.dot_general(
            emb_ref[...], wih_ref[j],
            (((1,), (1,)), ((), ())),
            preferred_element_type=jnp.float32,
        ) + bih_ref[j][None, :] + bhh_ref[j][None, :]
        mask = (vrow % NCELL == j).astype(jnp.float32)
        acc = acc + mask * gj
    table_ref[...] = acc
    for j in range(NCELL):
        wbf_ref[pl.ds(j * HID, HID), :] = jnp.transpose(
            whh_ref[j]).astype(jnp.bfloat16)


UNROLL = 8


def _lstm_kernel(tok_ref, table_ref, wbf_ref, out_ref, h_ref, c_ref):
    t = pl.program_id(0)

    @pl.when(t == 0)
    def _init():
        h_ref[...] = jnp.zeros((BATCH, HID), jnp.float32)
        c_ref[...] = jnp.zeros((BATCH, HID), jnp.float32)

    h = h_ref[...]
    c = c_ref[...]
    vcol = jax.lax.broadcasted_iota(jnp.int32, (BATCH, VOCAB), 1)
    for u in range(UNROLL):
        tok = tok_ref[u]
        onehot = (tok == vcol).astype(jnp.float32)
        gates = jnp.dot(onehot, table_ref[...],
                        preferred_element_type=jnp.float32)

        assign = tok % NCELL
        hbig = jnp.concatenate(
            [h * (assign == j).astype(jnp.float32) for j in range(NCELL)],
            axis=1).astype(jnp.bfloat16)
        gates = gates + jnp.dot(hbig, wbf_ref[...],
                                preferred_element_type=jnp.float32)

        i_g = jax.nn.sigmoid(gates[:, 0 * HID:1 * HID])
        f_g = jax.nn.sigmoid(gates[:, 1 * HID:2 * HID])
        g_g = jnp.tanh(gates[:, 2 * HID:3 * HID])
        o_g = jax.nn.sigmoid(gates[:, 3 * HID:4 * HID])
        c = f_g * c + i_g * g_g
        h = o_g * jnp.tanh(c)
    h_ref[...] = h
    c_ref[...] = c

    @pl.when(t == SEQ // UNROLL - 1)
    def _emit():
        out_ref[:, 0:HID] = h
        out_ref[:, HID:2 * HID] = c


@jax.jit
def kernel(input, emb, W_ih, W_hh, b_ih, b_hh):
    tokens = jnp.swapaxes(input.astype(jnp.int32), 0, 1).reshape(SEQ, BATCH, 1)

    table, wbf = pl.pallas_call(
        _prep_kernel,
        out_shape=(
            jax.ShapeDtypeStruct((VOCAB, G4), jnp.float32),
            jax.ShapeDtypeStruct((NCELL * HID, G4), jnp.bfloat16),
        ),
    )(emb, W_ih, b_ih, b_hh, W_hh)

    out = pl.pallas_call(
        _lstm_kernel,
        grid=(SEQ // UNROLL,),
        in_specs=[
            pl.BlockSpec((UNROLL, BATCH, 1), lambda t: (t, 0, 0)),
            pl.BlockSpec((VOCAB, G4), lambda t: (0, 0)),
            pl.BlockSpec((NCELL * HID, G4), lambda t: (0, 0)),
        ],
        out_specs=pl.BlockSpec((BATCH, 2 * HID), lambda t: (0, 0)),
        out_shape=jax.ShapeDtypeStruct((BATCH, 2 * HID), jnp.float32),
        scratch_shapes=[
            pltpu.VMEM((BATCH, HID), jnp.float32),
            pltpu.VMEM((BATCH, HID), jnp.float32),
        ],
    )(tokens, table, wbf)
    return out


# speedup vs baseline: 1.6163x; 1.0765x over previous
import jax
import jax.numpy as jnp
from jax.experimental import pallas as pl
from jax.experimental.pallas import tpu as pltpu

NCELL = 4
VOCAB = 16
EMB = 512
HID = 512
BATCH = 64
SEQ = 64
G4 = 4 * HID
KDIM = NCELL * HID + VOCAB
UNROLL = 8


def _lstm_kernel(tok_ref, emb_ref, wih_ref, bih_ref, bhh_ref, whh_ref,
                 out_ref, h_ref, c_ref, wcomb_ref):
    t = pl.program_id(0)

    @pl.when(t == 0)
    def _prep():
        h_ref[...] = jnp.zeros((BATCH, HID), jnp.float32)
        c_ref[...] = jnp.zeros((BATCH, HID), jnp.float32)
        for j in range(NCELL):
            wcomb_ref[pl.ds(j * HID, HID), :] = jnp.transpose(
                whh_ref[j]).astype(jnp.bfloat16)
        vrow = jax.lax.broadcasted_iota(jnp.int32, (VOCAB, 1), 0)
        acc = jnp.zeros((VOCAB, G4), dtype=jnp.float32)
        for j in range(NCELL):
            gj = jax.lax.dot_general(
                emb_ref[...], wih_ref[j],
                (((1,), (1,)), ((), ())),
                preferred_element_type=jnp.float32,
            ) + bih_ref[j][None, :] + bhh_ref[j][None, :]
            mask = (vrow % NCELL == j).astype(jnp.float32)
            acc = acc + mask * gj
        wcomb_ref[pl.ds(NCELL * HID, VOCAB), :] = acc.astype(jnp.bfloat16)

    h = h_ref[...]
    c = c_ref[...]
    vcol = jax.lax.broadcasted_iota(jnp.int32, (BATCH, VOCAB), 1)
    wcomb = wcomb_ref[...]
    for u in range(UNROLL):
        tok = tok_ref[u]
        onehot = (tok == vcol).astype(jnp.bfloat16)
        assign = tok % NCELL
        hbig = jnp.concatenate(
            [(h * (assign == j).astype(jnp.float32)).astype(jnp.bfloat16)
             for j in range(NCELL)] + [onehot],
            axis=1)
        gates = jnp.dot(hbig, wcomb, preferred_element_type=jnp.float32)

        i_g = jax.nn.sigmoid(gates[:, 0 * HID:1 * HID])
        f_g = jax.nn.sigmoid(gates[:, 1 * HID:2 * HID])
        g_g = jnp.tanh(gates[:, 2 * HID:3 * HID])
        o_g = jax.nn.sigmoid(gates[:, 3 * HID:4 * HID])
        c = f_g * c + i_g * g_g
        h = o_g * jnp.tanh(c)
    h_ref[...] = h
    c_ref[...] = c

    @pl.when(t == SEQ // UNROLL - 1)
    def _emit():
        out_ref[:, 0:HID] = h
        out_ref[:, HID:2 * HID] = c


@jax.jit
def kernel(input, emb, W_ih, W_hh, b_ih, b_hh):
    tokens = jnp.swapaxes(input.astype(jnp.int32), 0, 1).reshape(SEQ, BATCH, 1)

    out = pl.pallas_call(
        _lstm_kernel,
        grid=(SEQ // UNROLL,),
        in_specs=[
            pl.BlockSpec((UNROLL, BATCH, 1), lambda t: (t, 0, 0)),
            pl.BlockSpec((VOCAB, EMB), lambda t: (0, 0)),
            pl.BlockSpec((NCELL, G4, EMB), lambda t: (0, 0, 0)),
            pl.BlockSpec((NCELL, G4), lambda t: (0, 0)),
            pl.BlockSpec((NCELL, G4), lambda t: (0, 0)),
            pl.BlockSpec((NCELL, G4, HID), lambda t: (0, 0, 0)),
        ],
        out_specs=pl.BlockSpec((BATCH, 2 * HID), lambda t: (0, 0)),
        out_shape=jax.ShapeDtypeStruct((BATCH, 2 * HID), jnp.float32),
        scratch_shapes=[
            pltpu.VMEM((BATCH, HID), jnp.float32),
            pltpu.VMEM((BATCH, HID), jnp.float32),
            pltpu.VMEM((KDIM, G4), jnp.bfloat16),
        ],
    )(tokens, emb, W_ih, b_ih, b_hh, W_hh)
    return out
